# baseline (speedup 1.0000x reference)
import functools

import jax
import jax.numpy as jnp
from jax import lax
from jax.experimental import pallas as pl
from jax.experimental.pallas import tpu as pltpu
from jax.experimental.pallas import tpu_sc as plsc


_NBLK = 4096


def _argmin_body(nsteps, posq_ref, post_ref, p2_ref, iota_ref, out_ref,
                 bestv_ref, besti_ref):
    step = pl.program_id(0)

    @pl.when(step == 0)
    def _init():
        bestv_ref[...] = jnp.full(bestv_ref.shape, jnp.inf, jnp.float32)
        besti_ref[...] = jnp.zeros(besti_ref.shape, jnp.int32)

    qd = posq_ref[...]
    p = post_ref[...]
    dot2 = lax.dot_general(qd, p, (((1,), (1,)), ((), ())),
                           preferred_element_type=jnp.float32)
    q2 = 0.25 * jnp.sum(qd * qd, axis=1, keepdims=True)
    p2 = p2_ref[...]
    dist2 = q2 + p2 - dot2

    m = jnp.min(dist2, axis=1, keepdims=True)
    ii = iota_ref[...]
    loc = jnp.min(jnp.where(dist2 == m, ii, float(_NBLK)),
                  axis=1, keepdims=True)
    gidx = step * _NBLK + loc.astype(jnp.int32)

    better = m < bestv_ref[...]
    bestv_ref[...] = jnp.where(better, m, bestv_ref[...])
    besti_ref[...] = jnp.where(better, gidx, besti_ref[...])

    @pl.when(step == nsteps - 1)
    def _done():
        out_ref[...] = besti_ref[...]


def _nn_indices(pos2d, positions):
    q = pos2d.shape[0]
    n = positions.shape[0]
    npad = ((n + _NBLK - 1) // _NBLK) * _NBLK
    nsteps = npad // _NBLK
    posq = 2.0 * pos2d
    post = jnp.pad(positions, ((0, npad - n), (0, 0)),
                   constant_values=1e6)
    p2 = jnp.sum(post * post, axis=-1)[None, :]
    iota = lax.broadcasted_iota(jnp.float32, (1, _NBLK), 1)
    idx = pl.pallas_call(
        functools.partial(_argmin_body, nsteps),
        grid=(nsteps,),
        in_specs=[
            pl.BlockSpec((q, 3), lambda i: (0, 0)),
            pl.BlockSpec((_NBLK, 3), lambda i: (i, 0)),
            pl.BlockSpec((1, _NBLK), lambda i: (0, i)),
            pl.BlockSpec((1, _NBLK), lambda i: (0, 0)),
        ],
        out_specs=pl.BlockSpec((q, 1), lambda i: (0, 0)),
        out_shape=jax.ShapeDtypeStruct((q, 1), jnp.int32),
        scratch_shapes=[
            pltpu.VMEM((q, 1), jnp.float32),
            pltpu.VMEM((q, 1), jnp.int32),
        ],
    )(posq, post, p2, iota)
    return idx.reshape(q)


def _sc_gather(table, idx):
    b = idx.shape[0]
    d = table.shape[1]
    info = plsc.get_sparse_core_info()
    nc, ns = info.num_cores, info.num_subcores
    nw = nc * ns
    b_per_w = b // nw
    mesh = plsc.VectorSubcoreMesh(core_axis_name="c", subcore_axis_name="s")

    @functools.partial(
        pl.kernel,
        mesh=mesh,
        out_type=jax.ShapeDtypeStruct((b, d), jnp.float32),
        scratch_types=[
            pltpu.VMEM((b_per_w,), jnp.int32),
            pltpu.VMEM((b_per_w, d), jnp.float32),
            pltpu.SemaphoreType.DMA,
        ],
    )
    def gather_kernel(table_hbm, idx_hbm, out_hbm, idx_v, rows_v, sem):
        wid = lax.axis_index("s") * nc + lax.axis_index("c")
        base = wid * b_per_w
        pltpu.sync_copy(idx_hbm.at[pl.ds(base, b_per_w)], idx_v)
        pltpu.async_copy(table_hbm.at[idx_v], rows_v, sem).wait()
        pltpu.sync_copy(rows_v, out_hbm.at[pl.ds(base, b_per_w)])

    return gather_kernel(table, idx)


def _add_body(rows_ref, temp_ref, out_ref):
    rows = rows_ref[...]
    temp = temp_ref[...]
    out_ref[...] = rows[:, None, :] + temp[None, :, :]


def _temporal_add(rows, temporal):
    q, e = rows.shape
    t = temporal.shape[0]
    qb = 128
    return pl.pallas_call(
        _add_body,
        grid=(q // qb,),
        in_specs=[
            pl.BlockSpec((qb, e), lambda i: (i, 0)),
            pl.BlockSpec((t, e), lambda i: (0, 0)),
        ],
        out_specs=pl.BlockSpec((qb, t, e), lambda i: (i, 0, 0)),
        out_shape=jax.ShapeDtypeStruct((q, t, e), jnp.float32),
    )(rows, temporal)


def kernel(pos, positions, spatial_table, temporal_table):
    b, c, _ = pos.shape
    t = temporal_table.shape[0]
    e = spatial_table.shape[1]
    q = b * c
    idx = _nn_indices(pos.reshape(q, 3), positions)
    rows = _sc_gather(spatial_table, idx)
    pe = _temporal_add(rows, temporal_table)
    return pe.reshape(b, c * t, e)


# speedup vs baseline: 1.3387x; 1.3387x over previous
import functools

import jax
import jax.numpy as jnp
from jax import lax
from jax.experimental import pallas as pl
from jax.experimental.pallas import tpu as pltpu
from jax.experimental.pallas import tpu_sc as plsc


_NBLK = 4096


def _argmin_body(nsteps, posq_ref, post_ref, iota_ref, out_ref,
                 bestv_ref, besti_ref):
    step = pl.program_id(0)

    @pl.when(step == 0)
    def _init():
        bestv_ref[...] = jnp.full(bestv_ref.shape, jnp.inf, jnp.float32)
        besti_ref[...] = jnp.zeros(besti_ref.shape, jnp.int32)

    qd = posq_ref[...]
    p = post_ref[...]
    dot2 = jnp.dot(qd, p, preferred_element_type=jnp.float32)
    q2 = 0.25 * jnp.sum(qd * qd, axis=1, keepdims=True)
    p2 = jnp.sum(p * p, axis=0, keepdims=True)
    dist2 = q2 + p2 - dot2

    m = jnp.min(dist2, axis=1, keepdims=True)
    ii = iota_ref[...]
    loc = jnp.min(jnp.where(dist2 == m, ii, float(_NBLK)),
                  axis=1, keepdims=True)
    gidx = step * _NBLK + loc.astype(jnp.int32)

    better = m < bestv_ref[...]
    bestv_ref[...] = jnp.where(better, m, bestv_ref[...])
    besti_ref[...] = jnp.where(better, gidx, besti_ref[...])

    @pl.when(step == nsteps - 1)
    def _done():
        out_ref[...] = besti_ref[...]


def _nn_indices(pos2d, positions):
    q = pos2d.shape[0]
    n = positions.shape[0]
    npad = ((n + _NBLK - 1) // _NBLK) * _NBLK
    nsteps = npad // _NBLK
    posq = jnp.pad(2.0 * pos2d, ((0, 0), (0, 5)))
    post = jnp.pad(jnp.pad(positions, ((0, npad - n), (0, 0)),
                           constant_values=1e6),
                   ((0, 0), (0, 5))).T
    iota = lax.broadcasted_iota(jnp.float32, (1, _NBLK), 1)
    idx = pl.pallas_call(
        functools.partial(_argmin_body, nsteps),
        grid=(nsteps,),
        in_specs=[
            pl.BlockSpec((q, 8), lambda i: (0, 0)),
            pl.BlockSpec((8, _NBLK), lambda i: (0, i)),
            pl.BlockSpec((1, _NBLK), lambda i: (0, 0)),
        ],
        out_specs=pl.BlockSpec((q, 1), lambda i: (0, 0)),
        out_shape=jax.ShapeDtypeStruct((q, 1), jnp.int32),
        scratch_shapes=[
            pltpu.VMEM((q, 1), jnp.float32),
            pltpu.VMEM((q, 1), jnp.int32),
        ],
    )(posq, post, iota)
    return idx.reshape(q)


def _sc_gather(table, idx):
    b = idx.shape[0]
    d = table.shape[1]
    info = plsc.get_sparse_core_info()
    nc, ns = info.num_cores, info.num_subcores
    nw = nc * ns
    b_per_w = b // nw
    mesh = plsc.VectorSubcoreMesh(core_axis_name="c", subcore_axis_name="s")

    @functools.partial(
        pl.kernel,
        mesh=mesh,
        out_type=jax.ShapeDtypeStruct((b, d), jnp.float32),
        scratch_types=[
            pltpu.VMEM((b_per_w,), jnp.int32),
            pltpu.VMEM((b_per_w, d), jnp.float32),
            pltpu.SemaphoreType.DMA,
        ],
    )
    def gather_kernel(table_hbm, idx_hbm, out_hbm, idx_v, rows_v, sem):
        wid = lax.axis_index("s") * nc + lax.axis_index("c")
        base = wid * b_per_w
        pltpu.sync_copy(idx_hbm.at[pl.ds(base, b_per_w)], idx_v)
        pltpu.async_copy(table_hbm.at[idx_v], rows_v, sem).wait()
        pltpu.sync_copy(rows_v, out_hbm.at[pl.ds(base, b_per_w)])

    return gather_kernel(table, idx)


def _add_body(rows_ref, temp_ref, out_ref):
    rows = rows_ref[...]
    temp = temp_ref[...]
    out_ref[...] = rows[:, None, :] + temp[None, :, :]


def _temporal_add(rows, temporal):
    q, e = rows.shape
    t = temporal.shape[0]
    qb = 128
    return pl.pallas_call(
        _add_body,
        grid=(q // qb,),
        in_specs=[
            pl.BlockSpec((qb, e), lambda i: (i, 0)),
            pl.BlockSpec((t, e), lambda i: (0, 0)),
        ],
        out_specs=pl.BlockSpec((qb, t, e), lambda i: (i, 0, 0)),
        out_shape=jax.ShapeDtypeStruct((q, t, e), jnp.float32),
    )(rows, temporal)


def kernel(pos, positions, spatial_table, temporal_table):
    b, c, _ = pos.shape
    t = temporal_table.shape[0]
    e = spatial_table.shape[1]
    q = b * c
    idx = _nn_indices(pos.reshape(q, 3), positions)
    rows = _sc_gather(spatial_table, idx)
    pe = _temporal_add(rows, temporal_table)
    return pe.reshape(b, c * t, e)
